# baseline (speedup 1.0000x reference)
import functools

import jax
import jax.numpy as jnp
from jax import lax
from jax.experimental import pallas as pl
from jax.experimental.pallas import tpu as pltpu
from jax.experimental.pallas import tpu_sc as plsc

B = 16384
H = 50
D = 64
O = 64
NC = 2
NS = 16
NW = NC * NS
RPW = B // NW
NBUF = 4
HP = 64
NSLICE = D // 16


def _pooled_sc(x, emb):
    mesh = plsc.VectorSubcoreMesh(core_axis_name="c", subcore_axis_name="s")

    @functools.partial(
        pl.kernel,
        mesh=mesh,
        out_type=jax.ShapeDtypeStruct((B, D), jnp.float32),
        compiler_params=pltpu.CompilerParams(use_tc_tiling_on_sc=False),
        scratch_types=[
            pltpu.VMEM((RPW, HP), jnp.int32),
            pltpu.VMEM((NBUF, HP, D), jnp.float32),
            pltpu.VMEM((RPW, D), jnp.float32),
            pltpu.SemaphoreType.DMA,
            pltpu.SemaphoreType.DMA,
            pltpu.SemaphoreType.DMA,
            pltpu.SemaphoreType.DMA,
        ],
    )
    def k(x_hbm, emb_hbm, out_hbm, idx_v, buf_v, out_v, *sems):
        wid = lax.axis_index("s") * NC + lax.axis_index("c")
        pltpu.sync_copy(x_hbm.at[pl.ds(wid * RPW, RPW)], idx_v)

        def start(row, s):
            pltpu.async_copy(emb_hbm.at[idx_v.at[row]], buf_v.at[s], sems[s])

        def wait(s):
            pltpu.make_async_copy(
                emb_hbm.at[idx_v.at[0]], buf_v.at[s], sems[s]
            ).wait()

        for s in range(NBUF):
            start(s, s)

        def body(i, carry):
            for s in range(NBUF):
                r = i * NBUF + s
                wait(s)
                accs = [buf_v[s, 0, pl.ds(j * 16, 16)] for j in range(NSLICE)]
                for l in range(1, H):
                    for j in range(NSLICE):
                        accs[j] = accs[j] + buf_v[s, l, pl.ds(j * 16, 16)]

                @pl.when(r + NBUF < RPW)
                def _():
                    start(r + NBUF, s)

                for j in range(NSLICE):
                    out_v[r, pl.ds(j * 16, 16)] = accs[j] * (1.0 / H)
            return carry

        lax.fori_loop(0, RPW // NBUF, body, 0)
        pltpu.sync_copy(out_v, out_hbm.at[pl.ds(wid * RPW, RPW)])

    return k(x, emb)


def _linear_tc(pooled, W, b):
    BM = 2048

    def mm(p_ref, w_ref, b_ref, o_ref):
        o_ref[...] = (
            jnp.dot(p_ref[...], w_ref[...], preferred_element_type=jnp.float32)
            + b_ref[...]
        )

    return pl.pallas_call(
        mm,
        grid=(B // BM,),
        in_specs=[
            pl.BlockSpec((BM, D), lambda i: (i, 0)),
            pl.BlockSpec((D, O), lambda i: (0, 0)),
            pl.BlockSpec((1, O), lambda i: (0, 0)),
        ],
        out_specs=pl.BlockSpec((BM, O), lambda i: (i, 0)),
        out_shape=jax.ShapeDtypeStruct((B, O), jnp.float32),
    )(pooled, W, b.reshape(1, O))


def kernel(x, emb, W, b):
    xp = jnp.pad(x.astype(jnp.int32), ((0, 0), (0, HP - H)))
    pooled = _pooled_sc(xp, emb)
    return _linear_tc(pooled, W, b)


# speedup vs baseline: 6.5275x; 6.5275x over previous
import functools

import jax
import jax.numpy as jnp
from jax import lax
from jax.experimental import pallas as pl
from jax.experimental.pallas import tpu as pltpu
from jax.experimental.pallas import tpu_sc as plsc

B = 16384
H = 50
D = 64
O = 64
NC = 2
NS = 16
NW = NC * NS
RPW = B // NW
NBUF = 4
HP = 64
GL = 56
NSLICE = D // 16


def _pooled_sc(x, emb):
    mesh = plsc.VectorSubcoreMesh(core_axis_name="c", subcore_axis_name="s")

    @functools.partial(
        pl.kernel,
        mesh=mesh,
        out_type=jax.ShapeDtypeStruct((B, D), jnp.float32),
        compiler_params=pltpu.CompilerParams(use_tc_tiling_on_sc=False),
        scratch_types=[
            pltpu.VMEM((RPW, HP), jnp.int32),
            pltpu.VMEM((NBUF, GL, D), jnp.float32),
            pltpu.VMEM((RPW, D), jnp.float32),
            pltpu.SemaphoreType.DMA,
            pltpu.SemaphoreType.DMA,
            pltpu.SemaphoreType.DMA,
            pltpu.SemaphoreType.DMA,
        ],
    )
    def k(x_hbm, emb_hbm, out_hbm, idx_v, buf_v, out_v, *sems):
        wid = lax.axis_index("s") * NC + lax.axis_index("c")
        pltpu.sync_copy(x_hbm.at[pl.ds(wid * RPW, RPW)], idx_v)

        def start(row, s):
            pltpu.async_copy(
                emb_hbm.at[idx_v.at[row, pl.ds(0, GL)]], buf_v.at[s], sems[s]
            )

        def wait(s):
            pltpu.make_async_copy(
                emb_hbm.at[idx_v.at[0, pl.ds(0, GL)]], buf_v.at[s], sems[s]
            ).wait()

        for s in range(NBUF):
            start(s, s)

        def body(i, carry):
            for s in range(NBUF):
                r = i * NBUF + s
                wait(s)
                accs = [buf_v[s, 0, pl.ds(j * 16, 16)] for j in range(NSLICE)]
                for l in range(1, H):
                    for j in range(NSLICE):
                        accs[j] = accs[j] + buf_v[s, l, pl.ds(j * 16, 16)]

                @pl.when(r + NBUF < RPW)
                def _():
                    start(r + NBUF, s)

                for j in range(NSLICE):
                    out_v[r, pl.ds(j * 16, 16)] = accs[j] * (1.0 / H)
            return carry

        lax.fori_loop(0, RPW // NBUF, body, 0)
        pltpu.sync_copy(out_v, out_hbm.at[pl.ds(wid * RPW, RPW)])

    return k(x, emb)


def _linear_tc(pooled, W, b):
    BM = 2048

    def mm(p_ref, w_ref, b_ref, o_ref):
        o_ref[...] = (
            jnp.dot(p_ref[...], w_ref[...], preferred_element_type=jnp.float32)
            + b_ref[...]
        )

    return pl.pallas_call(
        mm,
        grid=(B // BM,),
        in_specs=[
            pl.BlockSpec((BM, D), lambda i: (i, 0)),
            pl.BlockSpec((D, O), lambda i: (0, 0)),
            pl.BlockSpec((1, O), lambda i: (0, 0)),
        ],
        out_specs=pl.BlockSpec((BM, O), lambda i: (i, 0)),
        out_shape=jax.ShapeDtypeStruct((B, O), jnp.float32),
    )(pooled, W, b.reshape(1, O))


def kernel(x, emb, W, b):
    xp = jnp.pad(x.astype(jnp.int32), ((0, 0), (0, HP - H)), mode="wrap")
    pooled = _pooled_sc(xp, emb)
    return _linear_tc(pooled, W, b)
